# baseline (speedup 1.0000x reference)
import jax
import jax.numpy as jnp
from jax.experimental import pallas as pl
from jax.experimental.pallas import tpu as pltpu

N_RNA = 1024
N_DIS = 512
HIDDEN = 128


def _mm_bf(a_bf, b_bf):
    return jax.lax.dot_general(a_bf, b_bf, (((1,), (0,)), ((), ())),
                               preferred_element_type=jnp.float32)


def _mmT_bf(a_bf, b_bf):
    return jax.lax.dot_general(a_bf, b_bf, (((0,), (0,)), ((), ())),
                               preferred_element_type=jnp.float32)


def _bf(v):
    return v.astype(jnp.bfloat16)


def _inv_deg(deg):
    return jnp.where(deg > 0, 1.0 / jnp.where(deg > 0, deg, 1.0), 0.0)


def _conv_chain(adj_f32, xw, b1, W2, b2):
    Bd = jnp.sum(adj_f32, axis=1, keepdims=True, dtype=jnp.float32)
    Dd = jnp.sum(adj_f32, axis=0, keepdims=True, dtype=jnp.float32).T
    Binv = _inv_deg(Bd)
    Dinv = _inv_deg(Dd)
    adj = _bf(adj_f32)
    e1 = _mm_bf(adj, _bf(xw)) * Binv
    h = jnp.maximum(_mmT_bf(adj, _bf(e1)) * Dinv + b1[...], 0.0)
    e2 = _mm_bf(adj, _bf(_mm_bf(_bf(h), _bf(W2[...])))) * Binv
    return _mmT_bf(adj, _bf(e2)) * Dinv + b2[...]


def _fused(w1m_h, cf_h, mf_h, cgs_h, wlm_h, dss_h, w1d_h, dgs_h, wld_h, df_h,
           W2m, W2d, b1m, b2m, b1d, b2d, blm, bld,
           out_ref,
           w1m_v, cf_v, mf_v, cgs_v, wlm_v, dss_v, w1d_v, dgs_v, wld_v, df_v,
           sems):
    hbm = (w1m_h, cf_h, mf_h, cgs_h, wlm_h, dss_h, w1d_h, dgs_h, wld_h, df_h)
    vmem = (w1m_v, cf_v, mf_v, cgs_v, wlm_v, dss_v, w1d_v, dgs_v, wld_v, df_v)
    cps = [pltpu.make_async_copy(h, v, sems.at[i])
           for i, (h, v) in enumerate(zip(hbm, vmem))]
    for c in cps:
        c.start()
    cps[0].wait()
    cps[1].wait()
    w1m_b = _bf(w1m_v[...])
    x1b = _bf(cf_v[...])
    xw = _mm_bf(x1b, w1m_b[:N_RNA])
    cps[3].wait()
    x2b = _bf(cgs_v[...])
    xw = xw + _mm_bf(x2b, w1m_b[N_RNA:])
    cps[4].wait()
    wlm_b = _bf(wlm_v[...])
    o2m = _mm_bf(x1b, wlm_b[:N_RNA]) + _mm_bf(x2b, wlm_b[N_RNA:]) + blm[...]
    cps[2].wait()
    o1m = _conv_chain(mf_v[...], xw, b1m, W2m, b2m)
    out_ref[:N_RNA, :] = (o1m + o2m) * 0.5
    cps[5].wait()
    cps[6].wait()
    w1d_b = _bf(w1d_v[...])
    y1b = _bf(dss_v[...])
    xwd = _mm_bf(y1b, w1d_b[:N_DIS])
    cps[7].wait()
    y2b = _bf(dgs_v[...])
    xwd = xwd + _mm_bf(y2b, w1d_b[N_DIS:])
    cps[8].wait()
    wld_b = _bf(wld_v[...])
    o2d = _mm_bf(y1b, wld_b[:N_DIS]) + _mm_bf(y2b, wld_b[N_DIS:]) + bld[...]
    cps[9].wait()
    o1d = _conv_chain(df_v[...], xwd, b1d, W2d, b2d)
    out_ref[N_RNA:, :] = (o1d + o2d) * 0.5


def kernel(m_f, d_f, c_func, c_gs, d_ss, d_gs, W1m, b1m, W2m, b2m,
           W1d, b1d, W2d, b2d, Wlm, blm, Wld, bld):
    f32 = jnp.float32
    hbm_spec = pl.BlockSpec(memory_space=pltpu.MemorySpace.HBM)
    vmem_spec = pl.BlockSpec(memory_space=pltpu.MemorySpace.VMEM)
    call = pl.pallas_call(
        _fused,
        out_shape=jax.ShapeDtypeStruct((N_RNA + N_DIS, HIDDEN), f32),
        in_specs=[hbm_spec] * 10 + [vmem_spec] * 8,
        out_specs=vmem_spec,
        scratch_shapes=[
            pltpu.VMEM((2 * N_RNA, HIDDEN), f32),
            pltpu.VMEM((N_RNA, N_RNA), f32),
            pltpu.VMEM((N_RNA, N_RNA), f32),
            pltpu.VMEM((N_RNA, N_RNA), f32),
            pltpu.VMEM((2 * N_RNA, HIDDEN), f32),
            pltpu.VMEM((N_DIS, N_DIS), f32),
            pltpu.VMEM((2 * N_DIS, HIDDEN), f32),
            pltpu.VMEM((N_DIS, N_DIS), f32),
            pltpu.VMEM((2 * N_DIS, HIDDEN), f32),
            pltpu.VMEM((N_DIS, N_DIS), f32),
            pltpu.SemaphoreType.DMA((10,)),
        ],
    )
    return call(
        W1m, c_func, m_f, c_gs, Wlm, d_ss, W1d, d_gs, Wld, d_f,
        W2m, W2d, b1m.reshape(1, HIDDEN), b2m.reshape(1, HIDDEN),
        b1d.reshape(1, HIDDEN), b2d.reshape(1, HIDDEN),
        blm.reshape(1, HIDDEN), bld.reshape(1, HIDDEN))


# speedup vs baseline: 1.0571x; 1.0571x over previous
import jax
import jax.numpy as jnp
from jax.experimental import pallas as pl
from jax.experimental.pallas import tpu as pltpu

N_RNA = 1024
N_DIS = 512
HIDDEN = 128


def _mm_bf(a_bf, b_bf):
    return jax.lax.dot_general(a_bf, b_bf, (((1,), (0,)), ((), ())),
                               preferred_element_type=jnp.float32)


def _mmT_bf(a_bf, b_bf):
    return jax.lax.dot_general(a_bf, b_bf, (((0,), (0,)), ((), ())),
                               preferred_element_type=jnp.float32)


def _bf(v):
    return v.astype(jnp.bfloat16)


def _inv_deg(deg):
    return jnp.where(deg > 0, 1.0 / jnp.where(deg > 0, deg, 1.0), 0.0)


def _conv_chain(adj_f32, xw, b1, W2, b2):
    Bd = jnp.sum(adj_f32, axis=1, keepdims=True, dtype=jnp.float32)
    Dd = jnp.sum(adj_f32, axis=0, keepdims=True, dtype=jnp.float32).T
    Binv = _inv_deg(Bd)
    Dinv = _inv_deg(Dd)
    adj = _bf(adj_f32)
    e1 = _mm_bf(adj, _bf(xw)) * Binv
    h = jnp.maximum(_mmT_bf(adj, _bf(e1)) * Dinv + b1[...], 0.0)
    e2 = _mm_bf(adj, _bf(_mm_bf(_bf(h), _bf(W2[...])))) * Binv
    return _mmT_bf(adj, _bf(e2)) * Dinv + b2[...]


def _fused(w1m_h, cf_h, mf_h, cgs_h, wlm_h, dss_h, w1d_h, dgs_h, wld_h, df_h,
           W2m, W2d, b1m, b2m, b1d, b2d, blm, bld,
           out_ref,
           w1m_v, cf_v, mf_v, cgs_v, wlm_v, dss_v, w1d_v, dgs_v, wld_v, df_v,
           s0, s1, s2, s3, s4, s5, s6, s7, s8, s9):
    hbm = (w1m_h, cf_h, mf_h, cgs_h, wlm_h, dss_h, w1d_h, dgs_h, wld_h, df_h)
    vmem = (w1m_v, cf_v, mf_v, cgs_v, wlm_v, dss_v, w1d_v, dgs_v, wld_v, df_v)
    sems = (s0, s1, s2, s3, s4, s5, s6, s7, s8, s9)
    cps = [pltpu.make_async_copy(h, v, s)
           for h, v, s in zip(hbm, vmem, sems)]
    for i in (0, 1, 3):
        cps[i].start()
    cps[0].wait()
    cps[1].wait()
    for i in (2, 4):
        cps[i].start()
    w1m_b = _bf(w1m_v[...])
    x1b = _bf(cf_v[...])
    xw = _mm_bf(x1b, w1m_b[:N_RNA])
    cps[3].wait()
    for i in (5, 6, 7, 8, 9):
        cps[i].start()
    x2b = _bf(cgs_v[...])
    xw = xw + _mm_bf(x2b, w1m_b[N_RNA:])
    cps[4].wait()
    wlm_b = _bf(wlm_v[...])
    o2m = _mm_bf(x1b, wlm_b[:N_RNA]) + _mm_bf(x2b, wlm_b[N_RNA:]) + blm[...]
    cps[2].wait()
    o1m = _conv_chain(mf_v[...], xw, b1m, W2m, b2m)
    out_ref[:N_RNA, :] = (o1m + o2m) * 0.5
    cps[5].wait()
    cps[6].wait()
    w1d_b = _bf(w1d_v[...])
    y1b = _bf(dss_v[...])
    xwd = _mm_bf(y1b, w1d_b[:N_DIS])
    cps[7].wait()
    y2b = _bf(dgs_v[...])
    xwd = xwd + _mm_bf(y2b, w1d_b[N_DIS:])
    cps[8].wait()
    wld_b = _bf(wld_v[...])
    o2d = _mm_bf(y1b, wld_b[:N_DIS]) + _mm_bf(y2b, wld_b[N_DIS:]) + bld[...]
    cps[9].wait()
    o1d = _conv_chain(df_v[...], xwd, b1d, W2d, b2d)
    out_ref[N_RNA:, :] = (o1d + o2d) * 0.5


def kernel(m_f, d_f, c_func, c_gs, d_ss, d_gs, W1m, b1m, W2m, b2m,
           W1d, b1d, W2d, b2d, Wlm, blm, Wld, bld):
    f32 = jnp.float32
    hbm_spec = pl.BlockSpec(memory_space=pltpu.MemorySpace.HBM)
    vmem_spec = pl.BlockSpec(memory_space=pltpu.MemorySpace.VMEM)
    call = pl.pallas_call(
        _fused,
        out_shape=jax.ShapeDtypeStruct((N_RNA + N_DIS, HIDDEN), f32),
        in_specs=[hbm_spec] * 10 + [vmem_spec] * 8,
        out_specs=vmem_spec,
        scratch_shapes=[
            pltpu.VMEM((2 * N_RNA, HIDDEN), f32),
            pltpu.VMEM((N_RNA, N_RNA), f32),
            pltpu.VMEM((N_RNA, N_RNA), f32),
            pltpu.VMEM((N_RNA, N_RNA), f32),
            pltpu.VMEM((2 * N_RNA, HIDDEN), f32),
            pltpu.VMEM((N_DIS, N_DIS), f32),
            pltpu.VMEM((2 * N_DIS, HIDDEN), f32),
            pltpu.VMEM((N_DIS, N_DIS), f32),
            pltpu.VMEM((2 * N_DIS, HIDDEN), f32),
            pltpu.VMEM((N_DIS, N_DIS), f32),
        ] + [pltpu.SemaphoreType.DMA] * 10,
    )
    return call(
        W1m, c_func, m_f, c_gs, Wlm, d_ss, W1d, d_gs, Wld, d_f,
        W2m, W2d, b1m.reshape(1, HIDDEN), b2m.reshape(1, HIDDEN),
        b1d.reshape(1, HIDDEN), b2d.reshape(1, HIDDEN),
        blm.reshape(1, HIDDEN), bld.reshape(1, HIDDEN))


# speedup vs baseline: 1.1013x; 1.0418x over previous
import jax
import jax.numpy as jnp
from jax.experimental import pallas as pl
from jax.experimental.pallas import tpu as pltpu

N_RNA = 1024
N_DIS = 512
HIDDEN = 128
CH = 512


def _mm(a, b):
    return jax.lax.dot_general(a, b, (((1,), (0,)), ((), ())),
                               preferred_element_type=jnp.float32)


def _mmT(a, b):
    return jax.lax.dot_general(a, b, (((0,), (0,)), ((), ())),
                               preferred_element_type=jnp.float32)


def _bf(v):
    return v.astype(jnp.bfloat16)


def _inv_deg(deg):
    return jnp.where(deg > 0, 1.0 / jnp.where(deg > 0, deg, 1.0), 0.0)


def _conv_chain(adj_f32, xw, b1, W2, b2):
    Bd = jnp.sum(adj_f32, axis=1, keepdims=True, dtype=jnp.float32)
    Dd = jnp.sum(adj_f32, axis=0, keepdims=True, dtype=jnp.float32).T
    Binv = _inv_deg(Bd)
    Dinv = _inv_deg(Dd)
    adj = _bf(adj_f32)
    e1 = _mm(adj, _bf(xw)) * Binv
    h = jnp.maximum(_mmT(adj, _bf(e1)) * Dinv + b1[...], 0.0)
    e2 = _mm(adj, _bf(_mm(_bf(h), _bf(W2[...])))) * Binv
    return _mmT(adj, _bf(e2)) * Dinv + b2[...]


def _fused(w1m_h, cf_h, cgs_h, wlm_h, mf_h, dss_h, w1d_h, dgs_h, wld_h, df_h,
           W2m, W2d, b1m, b2m, b1d, b2d, blm, bld,
           out_ref,
           w1m_v, cf_v, cgs_v, wlm_v, mf_v, dss_v, w1d_v, dgs_v, wld_v, df_v,
           *sems):
    nm = N_RNA // CH
    cp_w1m = pltpu.make_async_copy(w1m_h, w1m_v, sems[0])
    cp_cf = [pltpu.make_async_copy(cf_h.at[i * CH:(i + 1) * CH],
                                   cf_v.at[i * CH:(i + 1) * CH], sems[1 + i])
             for i in range(nm)]
    cp_cgs = [pltpu.make_async_copy(cgs_h.at[i * CH:(i + 1) * CH],
                                    cgs_v.at[i * CH:(i + 1) * CH],
                                    sems[1 + nm + i])
              for i in range(nm)]
    cp_wlm = pltpu.make_async_copy(wlm_h, wlm_v, sems[1 + 2 * nm])
    cp_mf = [pltpu.make_async_copy(mf_h.at[i * CH:(i + 1) * CH],
                                   mf_v.at[i * CH:(i + 1) * CH],
                                   sems[2 + 2 * nm + i])
             for i in range(nm)]
    base = 2 + 3 * nm
    cp_dss = pltpu.make_async_copy(dss_h, dss_v, sems[base])
    cp_w1d = pltpu.make_async_copy(w1d_h, w1d_v, sems[base + 1])
    cp_dgs = pltpu.make_async_copy(dgs_h, dgs_v, sems[base + 2])
    cp_wld = pltpu.make_async_copy(wld_h, wld_v, sems[base + 3])
    cp_df = pltpu.make_async_copy(df_h, df_v, sems[base + 4])

    cp_w1m.start()
    for i in range(nm):
        cp_cf[i].start()
        cp_cgs[i].start()
    cp_wlm.start()
    for c in cp_mf:
        c.start()
    cp_dss.start()
    cp_w1d.start()
    cp_dgs.start()
    cp_wld.start()
    cp_df.start()

    cp_w1m.wait()
    w1b = _bf(w1m_v[...])
    cp_wlm.wait()
    wlb = _bf(wlm_v[...])
    xw_parts, o2_parts = [], []
    for i in range(nm):
        cp_cf[i].wait()
        cp_cgs[i].wait()
        x1b = _bf(cf_v[i * CH:(i + 1) * CH])
        x2b = _bf(cgs_v[i * CH:(i + 1) * CH])
        xw_parts.append(_mm(x1b, w1b[:N_RNA]) + _mm(x2b, w1b[N_RNA:]))
        o2_parts.append(_mm(x1b, wlb[:N_RNA]) + _mm(x2b, wlb[N_RNA:]))
    xwb = _bf(jnp.concatenate(xw_parts, axis=0))
    adjb, binv, acc1, dd = [], [], 0.0, 0.0
    for i in range(nm):
        cp_mf[i].wait()
        ch = mf_v[i * CH:(i + 1) * CH]
        a = _bf(ch)
        adjb.append(a)
        bi = _inv_deg(jnp.sum(ch, axis=1, keepdims=True, dtype=jnp.float32))
        binv.append(bi)
        dd = dd + jnp.sum(ch, axis=0, keepdims=True, dtype=jnp.float32)
        acc1 = acc1 + _mmT(a, _bf(_mm(a, xwb) * bi))
    dinv = _inv_deg(dd).T
    h = jnp.maximum(acc1 * dinv + b1m[...], 0.0)
    xw2b = _bf(_mm(_bf(h), _bf(W2m[...])))
    acc2 = 0.0
    for i in range(nm):
        acc2 = acc2 + _mmT(adjb[i], _bf(_mm(adjb[i], xw2b) * binv[i]))
    o1m = acc2 * dinv + b2m[...]
    o2m = jnp.concatenate(o2_parts, axis=0) + blm[...]
    out_ref[:N_RNA, :] = (o1m + o2m) * 0.5

    cp_dss.wait()
    cp_w1d.wait()
    w1d_b = _bf(w1d_v[...])
    y1b = _bf(dss_v[...])
    xwd = _mm(y1b, w1d_b[:N_DIS])
    cp_dgs.wait()
    y2b = _bf(dgs_v[...])
    xwd = xwd + _mm(y2b, w1d_b[N_DIS:])
    cp_wld.wait()
    wld_b = _bf(wld_v[...])
    o2d = _mm(y1b, wld_b[:N_DIS]) + _mm(y2b, wld_b[N_DIS:]) + bld[...]
    cp_df.wait()
    o1d = _conv_chain(df_v[...], xwd, b1d, W2d, b2d)
    out_ref[N_RNA:, :] = (o1d + o2d) * 0.5


def kernel(m_f, d_f, c_func, c_gs, d_ss, d_gs, W1m, b1m, W2m, b2m,
           W1d, b1d, W2d, b2d, Wlm, blm, Wld, bld):
    f32 = jnp.float32
    nm = N_RNA // CH
    n_sems = 2 + 3 * nm + 5
    hbm_spec = pl.BlockSpec(memory_space=pltpu.MemorySpace.HBM)
    vmem_spec = pl.BlockSpec(memory_space=pltpu.MemorySpace.VMEM)
    call = pl.pallas_call(
        _fused,
        out_shape=jax.ShapeDtypeStruct((N_RNA + N_DIS, HIDDEN), f32),
        in_specs=[hbm_spec] * 10 + [vmem_spec] * 8,
        out_specs=vmem_spec,
        scratch_shapes=[
            pltpu.VMEM((2 * N_RNA, HIDDEN), f32),
            pltpu.VMEM((N_RNA, N_RNA), f32),
            pltpu.VMEM((N_RNA, N_RNA), f32),
            pltpu.VMEM((2 * N_RNA, HIDDEN), f32),
            pltpu.VMEM((N_RNA, N_RNA), f32),
            pltpu.VMEM((N_DIS, N_DIS), f32),
            pltpu.VMEM((2 * N_DIS, HIDDEN), f32),
            pltpu.VMEM((N_DIS, N_DIS), f32),
            pltpu.VMEM((2 * N_DIS, HIDDEN), f32),
            pltpu.VMEM((N_DIS, N_DIS), f32),
        ] + [pltpu.SemaphoreType.DMA] * n_sems,
    )
    return call(
        W1m, c_func, c_gs, Wlm, m_f, d_ss, W1d, d_gs, Wld, d_f,
        W2m, W2d, b1m.reshape(1, HIDDEN), b2m.reshape(1, HIDDEN),
        b1d.reshape(1, HIDDEN), b2d.reshape(1, HIDDEN),
        blm.reshape(1, HIDDEN), bld.reshape(1, HIDDEN))


# speedup vs baseline: 1.2590x; 1.1432x over previous
import jax
import jax.numpy as jnp
from jax.experimental import pallas as pl
from jax.experimental.pallas import tpu as pltpu

N_RNA = 1024
N_DIS = 512
HIDDEN = 128


def _mm(a, b):
    return jax.lax.dot_general(a, b, (((1,), (0,)), ((), ())),
                               preferred_element_type=jnp.float32)


def _mmT(a, b):
    return jax.lax.dot_general(a, b, (((0,), (0,)), ((), ())),
                               preferred_element_type=jnp.float32)


def _bf(v):
    return v.astype(jnp.bfloat16)


def _inv_deg(deg):
    return jnp.where(deg > 0, 1.0 / jnp.where(deg > 0, deg, 1.0), 0.0)


def _conv_chain(adj_f32, xw, b1, W2, b2):
    Bd = jnp.sum(adj_f32, axis=1, keepdims=True, dtype=jnp.float32)
    Dd = jnp.sum(adj_f32, axis=0, keepdims=True, dtype=jnp.float32).T
    Binv = _inv_deg(Bd)
    Dinv = _inv_deg(Dd)
    adj = _bf(adj_f32)
    e1 = _mm(adj, _bf(xw)) * Binv
    h = jnp.maximum(_mmT(adj, _bf(e1)) * Dinv + b1[...], 0.0)
    e2 = _mm(adj, _bf(_mm(_bf(h), _bf(W2[...])))) * Binv
    return _mmT(adj, _bf(e2)) * Dinv + b2[...]


def _fused(mf_h, dss_h, w1d_h, dgs_h, wld_h, df_h,
           cf, cgs, W1m, Wlm, W2m, W2d, b1m, b2m, b1d, b2d, blm, bld,
           out_ref,
           mf_v, dss_v, w1d_v, dgs_v, wld_v, df_v,
           s0, s1, s2, s3, s4, s5):
    cp_mf = pltpu.make_async_copy(mf_h, mf_v, s0)
    cp_dss = pltpu.make_async_copy(dss_h, dss_v, s1)
    cp_w1d = pltpu.make_async_copy(w1d_h, w1d_v, s2)
    cp_dgs = pltpu.make_async_copy(dgs_h, dgs_v, s3)
    cp_wld = pltpu.make_async_copy(wld_h, wld_v, s4)
    cp_df = pltpu.make_async_copy(df_h, df_v, s5)
    for c in (cp_mf, cp_dss, cp_w1d, cp_dgs, cp_wld, cp_df):
        c.start()
    w1b = _bf(W1m[...])
    wlb = _bf(Wlm[...])
    x1b = _bf(cf[...])
    x2b = _bf(cgs[...])
    xw = _mm(x1b, w1b[:N_RNA]) + _mm(x2b, w1b[N_RNA:])
    o2m = _mm(x1b, wlb[:N_RNA]) + _mm(x2b, wlb[N_RNA:]) + blm[...]
    cp_mf.wait()
    o1m = _conv_chain(mf_v[...], xw, b1m, W2m, b2m)
    out_ref[:N_RNA, :] = (o1m + o2m) * 0.5
    cp_dss.wait()
    cp_w1d.wait()
    w1d_b = _bf(w1d_v[...])
    y1b = _bf(dss_v[...])
    xwd = _mm(y1b, w1d_b[:N_DIS])
    cp_dgs.wait()
    y2b = _bf(dgs_v[...])
    xwd = xwd + _mm(y2b, w1d_b[N_DIS:])
    cp_wld.wait()
    wld_b = _bf(wld_v[...])
    o2d = _mm(y1b, wld_b[:N_DIS]) + _mm(y2b, wld_b[N_DIS:]) + bld[...]
    cp_df.wait()
    o1d = _conv_chain(df_v[...], xwd, b1d, W2d, b2d)
    out_ref[N_RNA:, :] = (o1d + o2d) * 0.5


def kernel(m_f, d_f, c_func, c_gs, d_ss, d_gs, W1m, b1m, W2m, b2m,
           W1d, b1d, W2d, b2d, Wlm, blm, Wld, bld):
    f32 = jnp.float32
    hbm_spec = pl.BlockSpec(memory_space=pltpu.MemorySpace.HBM)
    vmem_spec = pl.BlockSpec(memory_space=pltpu.MemorySpace.VMEM)
    call = pl.pallas_call(
        _fused,
        out_shape=jax.ShapeDtypeStruct((N_RNA + N_DIS, HIDDEN), f32),
        in_specs=[hbm_spec] * 6 + [vmem_spec] * 12,
        out_specs=vmem_spec,
        scratch_shapes=[
            pltpu.VMEM((N_RNA, N_RNA), f32),
            pltpu.VMEM((N_DIS, N_DIS), f32),
            pltpu.VMEM((2 * N_DIS, HIDDEN), f32),
            pltpu.VMEM((N_DIS, N_DIS), f32),
            pltpu.VMEM((2 * N_DIS, HIDDEN), f32),
            pltpu.VMEM((N_DIS, N_DIS), f32),
        ] + [pltpu.SemaphoreType.DMA] * 6,
    )
    return call(
        m_f, d_ss, W1d, d_gs, Wld, d_f,
        c_func, c_gs, W1m, Wlm, W2m, W2d,
        b1m.reshape(1, HIDDEN), b2m.reshape(1, HIDDEN),
        b1d.reshape(1, HIDDEN), b2d.reshape(1, HIDDEN),
        blm.reshape(1, HIDDEN), bld.reshape(1, HIDDEN))


# speedup vs baseline: 1.3583x; 1.0788x over previous
import jax
import jax.numpy as jnp
from jax.experimental import pallas as pl
from jax.experimental.pallas import tpu as pltpu

N_RNA = 1024
N_DIS = 512
HIDDEN = 128


def _mm(a, b):
    return jax.lax.dot_general(a, b, (((1,), (0,)), ((), ())),
                               preferred_element_type=jnp.float32)


def _mmT(a, b):
    return jax.lax.dot_general(a, b, (((0,), (0,)), ((), ())),
                               preferred_element_type=jnp.float32)


def _bf(v):
    return v.astype(jnp.bfloat16)


def _inv_deg(deg):
    return jnp.where(deg > 0, 1.0 / jnp.where(deg > 0, deg, 1.0), 0.0)


def _conv_chain(adj_f32, xw, b1, W2, b2):
    Bd = jnp.sum(adj_f32, axis=1, keepdims=True, dtype=jnp.float32)
    Dd = jnp.sum(adj_f32, axis=0, keepdims=True, dtype=jnp.float32).T
    Binv = _inv_deg(Bd)
    Dinv = _inv_deg(Dd)
    adj = _bf(adj_f32)
    e1 = _mm(adj, _bf(xw)) * Binv
    h = jnp.maximum(_mmT(adj, _bf(e1)) * Dinv + b1[...], 0.0)
    e2 = _mm(adj, _bf(_mm(_bf(h), _bf(W2[...])))) * Binv
    return _mmT(adj, _bf(e2)) * Dinv + b2[...]


def _fused(mf_h, dss_h, w1d_h, dgs_h, wld_h, df_h,
           cf, cgs, W1m, Wlm, W2m, W2d, b1m, b2m, b1d, b2d, blm, bld,
           out_ref,
           mf_v, dss_v, w1d_v, dgs_v, wld_v, df_v, xw_s, o2_s,
           s0, s1, s2, s3, s4, s5):
    pid = pl.program_id(0)
    cp_mf = pltpu.make_async_copy(mf_h, mf_v, s0)
    cp_dss = pltpu.make_async_copy(dss_h, dss_v, s1)
    cp_w1d = pltpu.make_async_copy(w1d_h, w1d_v, s2)
    cp_dgs = pltpu.make_async_copy(dgs_h, dgs_v, s3)
    cp_wld = pltpu.make_async_copy(wld_h, wld_v, s4)
    cp_df = pltpu.make_async_copy(df_h, df_v, s5)

    @pl.when(pid == 0)
    def _():
        for c in (cp_mf, cp_dss, cp_w1d, cp_dgs, cp_wld, cp_df):
            c.start()

    w1b = _bf(W1m[...])
    wlb = _bf(Wlm[...])
    x1b = _bf(cf[...])
    x2b = _bf(cgs[...])
    half = pid * (N_RNA // 2)
    xw_s[pl.ds(half, N_RNA // 2), :] = (
        _mm(x1b, w1b[:N_RNA]) + _mm(x2b, w1b[N_RNA:]))
    o2_s[pl.ds(half, N_RNA // 2), :] = (
        _mm(x1b, wlb[:N_RNA]) + _mm(x2b, wlb[N_RNA:]))

    @pl.when(pid == 1)
    def _():
        cp_mf.wait()
        o1m = _conv_chain(mf_v[...], xw_s[...], b1m, W2m, b2m)
        out_ref[:N_RNA, :] = (o1m + o2_s[...] + blm[...]) * 0.5
        cp_dss.wait()
        cp_w1d.wait()
        w1d_b = _bf(w1d_v[...])
        y1b = _bf(dss_v[...])
        xwd = _mm(y1b, w1d_b[:N_DIS])
        cp_dgs.wait()
        y2b = _bf(dgs_v[...])
        xwd = xwd + _mm(y2b, w1d_b[N_DIS:])
        cp_wld.wait()
        wld_b = _bf(wld_v[...])
        o2d = _mm(y1b, wld_b[:N_DIS]) + _mm(y2b, wld_b[N_DIS:]) + bld[...]
        cp_df.wait()
        o1d = _conv_chain(df_v[...], xwd, b1d, W2d, b2d)
        out_ref[N_RNA:, :] = (o1d + o2d) * 0.5


def kernel(m_f, d_f, c_func, c_gs, d_ss, d_gs, W1m, b1m, W2m, b2m,
           W1d, b1d, W2d, b2d, Wlm, blm, Wld, bld):
    f32 = jnp.float32
    hbm_spec = pl.BlockSpec(memory_space=pltpu.MemorySpace.HBM)
    vmem_spec = pl.BlockSpec(memory_space=pltpu.MemorySpace.VMEM)
    half_spec = pl.BlockSpec((N_RNA // 2, N_RNA), lambda i: (i, 0))
    full = pl.BlockSpec((N_RNA + N_DIS, HIDDEN), lambda i: (0, 0))
    w_spec = lambda r: pl.BlockSpec((r, HIDDEN), lambda i: (0, 0))
    call = pl.pallas_call(
        _fused,
        grid=(2,),
        out_shape=jax.ShapeDtypeStruct((N_RNA + N_DIS, HIDDEN), f32),
        in_specs=[hbm_spec] * 6 + [half_spec, half_spec]
        + [w_spec(2 * N_RNA), w_spec(2 * N_RNA), w_spec(HIDDEN),
           w_spec(HIDDEN)] + [w_spec(1)] * 6,
        out_specs=full,
        scratch_shapes=[
            pltpu.VMEM((N_RNA, N_RNA), f32),
            pltpu.VMEM((N_DIS, N_DIS), f32),
            pltpu.VMEM((2 * N_DIS, HIDDEN), f32),
            pltpu.VMEM((N_DIS, N_DIS), f32),
            pltpu.VMEM((2 * N_DIS, HIDDEN), f32),
            pltpu.VMEM((N_DIS, N_DIS), f32),
            pltpu.VMEM((N_RNA, HIDDEN), f32),
            pltpu.VMEM((N_RNA, HIDDEN), f32),
        ] + [pltpu.SemaphoreType.DMA] * 6,
        compiler_params=pltpu.CompilerParams(
            dimension_semantics=("arbitrary",)),
    )
    return call(
        m_f, d_ss, W1d, d_gs, Wld, d_f,
        c_func, c_gs, W1m, Wlm, W2m, W2d,
        b1m.reshape(1, HIDDEN), b2m.reshape(1, HIDDEN),
        b1d.reshape(1, HIDDEN), b2d.reshape(1, HIDDEN),
        blm.reshape(1, HIDDEN), bld.reshape(1, HIDDEN))
